# baseline (speedup 1.0000x reference)
import jax
import jax.numpy as jnp
from jax import lax
from jax.experimental import pallas as pl
from jax.experimental.pallas import tpu as pltpu
from jax.experimental.pallas import tpu_sc as plsc

B = 16384
CAT = 26
DN = 13
VOC = 100001
DT = 32
NTOK = DN + CAT

NC = 2
NS = 16
NW = NC * NS
BPW = B // NW
C = 32
NCHUNK = BPW // C


def _tok_body(xcat_hbm, xnum_hbm, w_hbm, bias_hbm, tab_hbm, out_hbm,
              xidx_v, rows_v, all_v, xnum_v, w_v, bias_v, sem):
    wid = lax.axis_index("s") * NC + lax.axis_index("c")

    pltpu.sync_copy(w_hbm, w_v)
    pltpu.sync_copy(bias_hbm, bias_v)

    def chunk_body(ci, carry):
        gb = (wid * NCHUNK + ci) * C

        pltpu.sync_copy(xcat_hbm.at[:, pl.ds(gb, C)], xidx_v)

        cps = []
        for f in range(CAT):
            cp = pltpu.make_async_copy(
                tab_hbm.at[f].at[xidx_v.at[f]], rows_v.at[pl.ds(f * C, C)], sem)
            cp.start()
            cps.append(cp)

        pltpu.sync_copy(xnum_hbm.at[pl.ds(gb * 16, C * 16)], xnum_v)

        for cp in cps:
            cp.wait()

        def b_body(b, carry2):
            xrow = xnum_v[pl.ds(b * 16, 16)]
            for d in range(DN):
                xi = xrow[d]
                for h in range(DT // 16):
                    s = pl.ds(h * 16, 16)
                    all_v[b, d, s] = xi * w_v[d, s] + bias_v[d, s]
            for f in range(CAT):
                for h in range(DT // 16):
                    s = pl.ds(h * 16, 16)
                    all_v[b, DN + f, s] = rows_v[f * C + b, s] + bias_v[DN + f, s]
            return carry2

        lax.fori_loop(0, C, b_body, 0)

        pltpu.sync_copy(all_v, out_hbm.at[pl.ds(gb, C)])
        return carry

    lax.fori_loop(0, NCHUNK, chunk_body, 0)


def kernel(x_cat, x_num, weight, bias, tables):
    xcat_t = x_cat.astype(jnp.int32).T
    x_num16 = jnp.pad(x_num, ((0, 0), (0, 16 - DN))).reshape(B * 16)

    tok = pl.kernel(
        _tok_body,
        out_type=jax.ShapeDtypeStruct((B, NTOK, DT), jnp.float32),
        mesh=plsc.VectorSubcoreMesh(core_axis_name="c", subcore_axis_name="s"),
        compiler_params=pltpu.CompilerParams(use_tc_tiling_on_sc=False),
        scratch_types=[
            pltpu.VMEM((CAT, C), jnp.int32),
            pltpu.VMEM((CAT * C, DT), jnp.float32),
            pltpu.VMEM((C, NTOK, DT), jnp.float32),
            pltpu.VMEM((C * 16,), jnp.float32),
            pltpu.VMEM((DN, DT), jnp.float32),
            pltpu.VMEM((NTOK, DT), jnp.float32),
            pltpu.SemaphoreType.DMA,
        ],
    )
    return tok(xcat_t, x_num16, weight, bias, tables)


# speedup vs baseline: 2.2043x; 2.2043x over previous
import jax
import jax.numpy as jnp
from jax import lax
from jax.experimental import pallas as pl
from jax.experimental.pallas import tpu as pltpu
from jax.experimental.pallas import tpu_sc as plsc

B = 16384
CAT = 26
DN = 13
VOC = 100001
DT = 32
NTOK = DN + CAT

NC = 2
NS = 16
NW = NC * NS
BPW = B // NW
C = 32
NCHUNK = BPW // C


def _tok_body(xcat_hbm, xnum_hbm, w_hbm, bias_hbm, *rest):
    tabs = rest[:CAT]
    (out_hbm, xidx_v, rows_v, all_v, xnum_v, w_v, bias_v, sem) = rest[CAT:]
    wid = lax.axis_index("s") * NC + lax.axis_index("c")

    pltpu.sync_copy(w_hbm, w_v)
    pltpu.sync_copy(bias_hbm, bias_v)

    def chunk_body(ci, carry):
        gb = (wid * NCHUNK + ci) * C

        pltpu.sync_copy(xcat_hbm.at[:, pl.ds(gb, C)], xidx_v)

        cps = []
        for f in range(CAT):
            cp = pltpu.make_async_copy(
                tabs[f].at[xidx_v.at[f]], rows_v.at[pl.ds(f * C, C)], sem)
            cp.start()
            cps.append(cp)

        pltpu.sync_copy(xnum_hbm.at[pl.ds(gb * 16, C * 16)], xnum_v)

        for cp in cps:
            cp.wait()

        def b_body(b, carry2):
            xrow = xnum_v[pl.ds(b * 16, 16)]
            for d in range(DN):
                xi = xrow[d]
                for h in range(DT // 16):
                    s = pl.ds(h * 16, 16)
                    all_v[b, d, s] = xi * w_v[d, s] + bias_v[d, s]
            for f in range(CAT):
                for h in range(DT // 16):
                    s = pl.ds(h * 16, 16)
                    all_v[b, DN + f, s] = rows_v[f * C + b, s] + bias_v[DN + f, s]
            return carry2

        lax.fori_loop(0, C, b_body, 0)

        pltpu.sync_copy(all_v, out_hbm.at[pl.ds(gb, C)])
        return carry

    lax.fori_loop(0, NCHUNK, chunk_body, 0)


def kernel(x_cat, x_num, weight, bias, tables):
    xcat_t = x_cat.astype(jnp.int32).T
    x_num16 = jnp.pad(x_num, ((0, 0), (0, 16 - DN))).reshape(B * 16)

    tok = pl.kernel(
        _tok_body,
        out_type=jax.ShapeDtypeStruct((B, NTOK, DT), jnp.float32),
        mesh=plsc.VectorSubcoreMesh(core_axis_name="c", subcore_axis_name="s"),
        compiler_params=pltpu.CompilerParams(use_tc_tiling_on_sc=False),
        scratch_types=[
            pltpu.VMEM((CAT, C), jnp.int32),
            pltpu.VMEM((CAT * C, DT), jnp.float32),
            pltpu.VMEM((C, NTOK, DT), jnp.float32),
            pltpu.VMEM((C * 16,), jnp.float32),
            pltpu.VMEM((DN, DT), jnp.float32),
            pltpu.VMEM((NTOK, DT), jnp.float32),
            pltpu.SemaphoreType.DMA,
        ],
    )
    return tok(xcat_t, x_num16, weight, bias, *[tables[f] for f in range(CAT)])
